# baseline (speedup 1.0000x reference)
import functools
import jax
import jax.numpy as jnp
from jax import lax
from jax.experimental import pallas as pl
from jax.experimental.pallas import tpu as pltpu
from jax.experimental.pallas import tpu_sc as plsc

NC, NS, L = 2, 16, 16
NW = NC * NS
B = 8
N = 64 * 64 * 64
K = 16
ROWS_PER_W = N // NW
C = 2048
CHUNKS = B * (ROWS_PER_W // C)
CW = C * K


def _sc_body(x_hbm, hard_hbm, soft_hbm, xbuf, softbuf, hardbuf):
    wid = lax.axis_index("s") * NC + lax.axis_index("c")
    lane = lax.broadcasted_iota(jnp.int32, (L,), 0)

    def chunk_body(t, _):
        b = t // (ROWS_PER_W // C)
        c = t % (ROWS_PER_W // C)
        n0 = wid * ROWS_PER_W + c * C
        row0 = b * N + n0
        pltpu.sync_copy(x_hbm.at[pl.ds(row0 * K, CW)], xbuf)

        def group_body(g, _):
            base = g * (L * K)
            idx = [base + lane * K + k for k in range(K)]
            xs = [plsc.load_gather(xbuf, [idx[k]]) for k in range(K)]
            m = xs[0]
            best = jnp.zeros((L,), jnp.int32)
            for k in range(1, K):
                gt = xs[k] > m
                m = jnp.where(gt, xs[k], m)
                best = jnp.where(gt, k, best)
            es = [jnp.exp(xs[k] - m) for k in range(K)]
            s = es[0]
            for k in range(1, K):
                s = s + es[k]
            r = 1.0 / s
            for k in range(K):
                plsc.store_scatter(softbuf, [idx[k]], es[k] * r)
            for k in range(K):
                h = jnp.where(best == k, 1.0, 0.0)
                hardbuf[pl.ds(k * C + g * L, L)] = h
            return _

        lax.fori_loop(0, C // L, group_body, None)
        pltpu.sync_copy(softbuf, soft_hbm.at[pl.ds(row0 * K, CW)])
        for k in range(K):
            off = b * (K * N) + k * N + n0
            pltpu.sync_copy(hardbuf.at[pl.ds(k * C, C)],
                            hard_hbm.at[pl.ds(off, C)])
        return _

    lax.fori_loop(0, CHUNKS, chunk_body, None)


def kernel(mask):
    bshape = mask.shape
    x = mask.reshape(-1)

    sc_call = pl.kernel(
        _sc_body,
        out_type=[
            jax.ShapeDtypeStruct((B * K * N,), jnp.float32),
            jax.ShapeDtypeStruct((B * N * K,), jnp.float32),
        ],
        mesh=plsc.VectorSubcoreMesh(
            core_axis_name="c", subcore_axis_name="s",
            num_cores=NC, num_subcores=NS),
        scratch_types=[
            pltpu.VMEM((CW,), jnp.float32),
            pltpu.VMEM((CW,), jnp.float32),
            pltpu.VMEM((K * C,), jnp.float32),
        ],
        compiler_params=pltpu.CompilerParams(needs_layout_passes=False),
    )
    hard, soft = sc_call(x)
    hard_mask = hard.reshape(B, K, bshape[1], bshape[2], bshape[3])
    y_soft = soft.reshape(B * N, K)
    return (hard_mask, y_soft)
